# Initial kernel scaffold; baseline (speedup 1.0000x reference)
import jax
import jax.numpy as jnp
from jax.experimental import pallas as pl


def kernel(params, x_left, x_right, y, index1, index2, y1_context, y2_context):
    raise NotImplementedError("write your pallas kernel here")


# speedup vs baseline: 15.9324x; 15.9324x over previous
import functools

import jax
import jax.numpy as jnp
from jax import lax
from jax.experimental import pallas as pl
from jax.experimental.pallas import tpu as pltpu
from jax.experimental.pallas import tpu_sc as plsc

B = 256
T = 50
H = 64
EMB = 32
K = 20
SIZE = 100000
CHUNK = 2048
NCHUNK = (SIZE + CHUNK - 1) // CHUNK
PAD_SIZE = NCHUNK * CHUNK
NEG = -3.0e38


def _gru_body(x_ref, wih0_ref, whh0_ref, wih1_ref, whh1_ref, b0_ref, b1_ref,
              out_ref):
    wih0 = wih0_ref[0]
    whh0 = whh0_ref[0]
    wih1 = wih1_ref[0]
    whh1 = whh1_ref[0]
    b_ih0 = b0_ref[0, 0:1, :]
    b_hh0 = b0_ref[0, 1:2, :]
    b_ih1 = b1_ref[0, 0:1, :]
    b_hh1 = b1_ref[0, 1:2, :]

    def gru_cell(gx, h, whh, b_hh):
        gh = jnp.dot(h, whh, preferred_element_type=jnp.float32) + b_hh
        r = jax.nn.sigmoid(gx[:, :H] + gh[:, :H])
        z = jax.nn.sigmoid(gx[:, H:2 * H] + gh[:, H:2 * H])
        n = jnp.tanh(gx[:, 2 * H:] + r * gh[:, 2 * H:])
        return (1.0 - z) * n + z * h

    x = x_ref[0]
    iota_t = jax.lax.broadcasted_iota(jnp.int32, (T, 1), 0)

    def step(t, carry):
        h1, h2 = carry
        onehot = (iota_t == t).astype(jnp.float32)
        xt = jnp.dot(x, onehot, preferred_element_type=jnp.float32)
        gx0 = xt * wih0 + b_ih0
        h1 = gru_cell(gx0, h1, whh0, b_hh0)
        gx1 = jnp.dot(h1, wih1, preferred_element_type=jnp.float32) + b_ih1
        h2 = gru_cell(gx1, h2, whh1, b_hh1)
        return (h1, h2)

    h0 = jnp.zeros((B, H), jnp.float32)
    _, h2 = lax.fori_loop(0, T, step, (h0, h0))
    out_ref[0] = h2


def _run_gru(x2, wih0, whh0, wih1, whh1, b0, b1):
    return pl.pallas_call(
        _gru_body,
        grid=(2,),
        in_specs=[
            pl.BlockSpec((1, B, T), lambda i: (i, 0, 0)),
            pl.BlockSpec((1, 1, 3 * H), lambda i: (i, 0, 0)),
            pl.BlockSpec((1, H, 3 * H), lambda i: (i, 0, 0)),
            pl.BlockSpec((1, H, 3 * H), lambda i: (i, 0, 0)),
            pl.BlockSpec((1, H, 3 * H), lambda i: (i, 0, 0)),
            pl.BlockSpec((1, 2, 3 * H), lambda i: (i, 0, 0)),
            pl.BlockSpec((1, 2, 3 * H), lambda i: (i, 0, 0)),
        ],
        out_specs=pl.BlockSpec((1, B, H), lambda i: (i, 0, 0)),
        out_shape=jax.ShapeDtypeStruct((2, B, H), jnp.float32),
    )(x2, wih0, whh0, wih1, whh1, b0, b1)


def _topk_body(eblk_ref, off_ref, emb_ref, vals_ref, idxs_ref, s_ref, rv_ref,
               ri_ref):
    i = pl.program_id(0)

    @pl.when(i == 0)
    def _init():
        rv_ref[...] = jnp.full((B, 128), NEG, jnp.float32)
        ri_ref[...] = jnp.zeros((B, 128), jnp.int32)

    eb = eblk_ref[...]
    off = off_ref[...]
    e = jnp.zeros((B, EMB), jnp.float32)
    for kq in range(4):
        e = e + jnp.where(off == 32 * kq, eb[:, 32 * kq:32 * (kq + 1)], 0.0)
    emb_c = emb_ref[...]
    na = jnp.sum(e * e, axis=1, keepdims=True)
    nb = jnp.sum(emb_c * emb_c, axis=0, keepdims=True)
    s2 = jnp.dot(e, emb_c, preferred_element_type=jnp.float32)
    d2 = jnp.maximum(na + nb - 2.0 * s2, 0.0)
    col = jax.lax.broadcasted_iota(jnp.int32, (B, CHUNK), 1) + i * CHUNK
    s_ref[...] = jnp.where(col < SIZE, -d2, NEG)

    iota_c = jax.lax.broadcasted_iota(jnp.int32, (B, CHUNK), 1)
    iota_r = jax.lax.broadcasted_iota(jnp.int32, (B, 128), 1)
    big = jnp.int32(2 ** 30)

    vals = []
    idxs = []
    for _ in range(K + 1):
        s = s_ref[...]
        rv = rv_ref[...]
        ri = ri_ref[...]
        m_s = jnp.max(s, axis=1, keepdims=True)
        m_r = jnp.max(rv, axis=1, keepdims=True)
        take = m_s > m_r
        c_s = jnp.min(jnp.where(s == m_s, iota_c, big), axis=1, keepdims=True)
        c_r = jnp.min(jnp.where(rv == m_r, iota_r, big), axis=1, keepdims=True)
        g_r = jnp.sum(jnp.where(iota_r == c_r, ri, 0), axis=1, keepdims=True)
        s_ref[...] = jnp.where((iota_c == c_s) & take, NEG, s)
        rv_ref[...] = jnp.where((iota_r == c_r) & (~take), NEG, rv)
        vals.append(jnp.where(take, m_s, m_r))
        idxs.append(jnp.where(take, c_s + i * CHUNK, g_r))

    pad_v = jnp.full((B, 128 - (K + 1)), NEG, jnp.float32)
    pad_i = jnp.zeros((B, 128 - (K + 1)), jnp.int32)
    rv_new = jnp.concatenate(vals + [pad_v], axis=1)
    ri_new = jnp.concatenate(idxs + [pad_i], axis=1)
    rv_ref[...] = rv_new
    ri_ref[...] = ri_new

    @pl.when(i == NCHUNK - 1)
    def _emit():
        vals_ref[...] = rv_new[:, :K + 1]
        idxs_ref[...] = ri_new[:, :K + 1]


def _run_topk(e_blk, off4, emb_pad):
    return pl.pallas_call(
        _topk_body,
        grid=(NCHUNK,),
        in_specs=[
            pl.BlockSpec((B, 128), lambda i: (0, 0)),
            pl.BlockSpec((B, 1), lambda i: (0, 0)),
            pl.BlockSpec((EMB, CHUNK), lambda i: (0, i)),
        ],
        out_specs=[
            pl.BlockSpec((B, K + 1), lambda i: (0, 0)),
            pl.BlockSpec((B, K + 1), lambda i: (0, 0)),
        ],
        out_shape=[
            jax.ShapeDtypeStruct((B, K + 1), jnp.float32),
            jax.ShapeDtypeStruct((B, K + 1), jnp.int32),
        ],
        scratch_shapes=[
            pltpu.VMEM((B, CHUNK), jnp.float32),
            pltpu.VMEM((B, 128), jnp.float32),
            pltpu.VMEM((B, 128), jnp.int32),
        ],
    )(e_blk, off4, emb_pad)


_NW = 32
_BPW = B // _NW
NIDX = 24
_IPW = B * NIDX // _NW


def _sc_mesh():
    return plsc.VectorSubcoreMesh(core_axis_name="c", subcore_axis_name="s")


def _embsel_body(emb1, bidx1, emb2, bidx2, o1, o2, bidx_v, rows_v, sem):
    wid = lax.axis_index("s") * 2 + lax.axis_index("c")
    for emb, bidx, o in ((emb1, bidx1, o1), (emb2, bidx2, o2)):
        pltpu.sync_copy(bidx.at[pl.ds(wid * _BPW, _BPW)], bidx_v)
        pltpu.async_copy(emb.at[bidx_v], rows_v, sem).wait()
        pltpu.sync_copy(rows_v, o.at[pl.ds(wid * _BPW, _BPW)])


def _run_embsel(emb1_128, bidx1, emb2_128, bidx2):
    f = pl.kernel(
        _embsel_body, mesh=_sc_mesh(),
        out_type=[jax.ShapeDtypeStruct((B, 128), jnp.float32),
                  jax.ShapeDtypeStruct((B, 128), jnp.float32)],
        scratch_types=[pltpu.VMEM((_BPW,), jnp.int32),
                       pltpu.VMEM((_BPW, 128), jnp.float32),
                       pltpu.SemaphoreType.DMA],
    )
    return f(emb1_128, bidx1, emb2_128, bidx2)


def _ygather_body(y1, rows1, y2, rows2, o1, o2, ridx_v, rows_v, sem):
    wid = lax.axis_index("s") * 2 + lax.axis_index("c")
    base = wid * _IPW
    for y, rows, o in ((y1, rows1, o1), (y2, rows2, o2)):
        pltpu.sync_copy(rows.at[wid], ridx_v)
        for h in range(2):
            pltpu.async_copy(y.at[ridx_v.at[h]],
                             rows_v.at[pl.ds(96 * h, 96)], sem).wait()
        pltpu.sync_copy(rows_v, o.at[pl.ds(base, _IPW)])


def _run_ygather(y1_128, rows1, y2_128, rows2):
    f = pl.kernel(
        _ygather_body, mesh=_sc_mesh(),
        out_type=[jax.ShapeDtypeStruct((B * NIDX, 128), jnp.float32),
                  jax.ShapeDtypeStruct((B * NIDX, 128), jnp.float32)],
        scratch_types=[pltpu.VMEM((2, _IPW // 2), jnp.int32),
                       pltpu.VMEM((_IPW, 128), jnp.float32),
                       pltpu.SemaphoreType.DMA],
    )
    return f(y1_128, rows1, y2_128, rows2)


def _final_body(hh_ref, v1_ref, v2_ref, yb1_ref, ln1_ref, yb2_ref, ln2_ref,
                wmean_ref, wstd_ref, wout_ref, bout_ref, wmo_ref, wso_ref,
                bs_ref, y_ref, err1_ref, err2_ref, mo_ref):
    temp = jnp.concatenate([hh_ref[0], hh_ref[1]], axis=1)
    mean_ts = jnp.dot(temp, wmean_ref[...],
                      preferred_element_type=jnp.float32) + bs_ref[0, 0]
    std_ts = jnp.dot(temp, wstd_ref[...],
                     preferred_element_type=jnp.float32) + bs_ref[0, 1]

    iota3 = jax.lax.broadcasted_iota(jnp.int32, (B, NIDX, 128), 2)

    def feats_one(v_ref, yb_ref, ln_ref):
        d2 = -v_ref[...]
        w = jnp.exp(-jnp.where(d2 > 0, jnp.sqrt(jnp.where(d2 > 0, d2, 1.0)),
                               0.0))
        wsel = w[:, 1:]
        oh = (iota3 == ln_ref[...]).astype(jnp.float32)
        ys = jnp.sum(yb_ref[...] * oh, axis=2)
        sel = ys[:, 1:K + 1]
        ws = jnp.sum(wsel, axis=1, keepdims=True)
        wm = jnp.sum(sel * wsel, axis=1, keepdims=True) / ws
        mu = jnp.mean(sel, axis=1, keepdims=True)
        st = jnp.sqrt(jnp.sum((sel - mu) ** 2, axis=1, keepdims=True)
                      / (K - 1))
        return jnp.concatenate([wm, ws, st], axis=1)

    f1 = feats_one(v1_ref, yb1_ref, ln1_ref)
    f2 = feats_one(v2_ref, yb2_ref, ln2_ref)
    feats = jnp.concatenate([f1, f2, mean_ts, std_ts], axis=1)
    h = jnp.maximum(jnp.dot(feats, wout_ref[...],
                            preferred_element_type=jnp.float32)
                    + bout_ref[...], 0.0)
    mean_o = jnp.dot(h, wmo_ref[...],
                     preferred_element_type=jnp.float32) + bs_ref[0, 2]
    std_o = jnp.dot(h, wso_ref[...],
                    preferred_element_type=jnp.float32) + bs_ref[0, 3]

    yb = y_ref[...]
    e1 = ((yb - mean_ts) ** 2 / jnp.exp(std_ts)) + std_ts
    e2 = ((yb - mean_o) ** 2 / jnp.exp(std_o)) + std_o
    err1_ref[...] = jnp.sum(e1, axis=0, keepdims=True) / B
    err2_ref[...] = jnp.sum(e2, axis=0, keepdims=True) / B
    mo_ref[...] = mean_o


def _run_final(hh, v1, v2, yb1, ln1, yb2, ln2, wmean, wstd, wout, bout, wmo,
               wso, bs, y):
    return pl.pallas_call(
        _final_body,
        out_shape=[
            jax.ShapeDtypeStruct((1, 1), jnp.float32),
            jax.ShapeDtypeStruct((1, 1), jnp.float32),
            jax.ShapeDtypeStruct((B, 1), jnp.float32),
        ],
    )(hh, v1, v2, yb1, ln1, yb2, ln2, wmean, wstd, wout, bout, wmo,
      wso, bs, y)


def kernel(params, x_left, x_right, y, index1, index2, y1_context, y2_context):
    p = params
    x2 = jnp.stack([x_left, x_right])
    wih0 = jnp.stack([p['W_ih_left0'][:, 0][None, :],
                      p['W_ih_right0'][:, 0][None, :]])
    whh0 = jnp.stack([p['W_hh_left0'].T, p['W_hh_right0'].T])
    wih1 = jnp.stack([p['W_ih_left1'].T, p['W_ih_right1'].T])
    whh1 = jnp.stack([p['W_hh_left1'].T, p['W_hh_right1'].T])
    b0 = jnp.stack([jnp.stack([p['b_ih_left0'], p['b_hh_left0']]),
                    jnp.stack([p['b_ih_right0'], p['b_hh_right0']])])
    b1 = jnp.stack([jnp.stack([p['b_ih_left1'], p['b_hh_left1']]),
                    jnp.stack([p['b_ih_right1'], p['b_hh_right1']])])
    hh = _run_gru(x2, wih0, whh0, wih1, whh1, b0, b1)

    idx1i = index1.astype(jnp.int32)
    idx2i = index2.astype(jnp.int32)
    eb1, eb2 = _run_embsel(p['emb1'].reshape(-1, 128), idx1i // 4,
                           p['emb2'].reshape(-1, 128), idx2i // 4)
    off1 = ((idx1i % 4) * EMB)[:, None]
    off2 = ((idx2i % 4) * EMB)[:, None]

    emb1_pad = jnp.pad(p['emb1'].T, ((0, 0), (0, PAD_SIZE - SIZE)))
    emb2_pad = jnp.pad(p['emb2'].T, ((0, 0), (0, PAD_SIZE - SIZE)))
    v1, i1 = _run_topk(eb1, off1, emb1_pad)
    v2, i2 = _run_topk(eb2, off2, emb2_pad)

    bb = jnp.arange(B, dtype=jnp.int32)[:, None] * SIZE
    def _yidx(ii):
        fi = jnp.pad(bb + ii, ((0, 0), (0, NIDX - (K + 1))))
        rows = (fi // 128).reshape(_NW, 2, _IPW // 2)
        lanes = (fi % 128).reshape(B, NIDX, 1)
        return rows, lanes
    rows1, lanes1 = _yidx(i1)
    rows2, lanes2 = _yidx(i2)
    yb1, yb2 = _run_ygather(
        y1_context.reshape(B * SIZE // 128, 128), rows1,
        y2_context.reshape(B * SIZE // 128, 128), rows2)
    yb1 = yb1.reshape(B, NIDX, 128)
    yb2 = yb2.reshape(B, NIDX, 128)

    bs = jnp.stack([p['b_mean'][0], p['b_std'][0],
                    p['b_mo'][0], p['b_so'][0]])[None, :]
    err1, err2, mean_o = _run_final(
        hh, v1, v2, yb1, lanes1, yb2, lanes2,
        p['W_mean'].T, p['W_std'].T, p['W_out1'].T, p['b_out1'][None, :],
        p['W_mo'].T, p['W_so'].T, bs, y[:, None])
    return (err1.reshape(()), err2.reshape(()), mean_o)


# speedup vs baseline: 27.8483x; 1.7479x over previous
import functools

import jax
import jax.numpy as jnp
from jax import lax
from jax.experimental import pallas as pl
from jax.experimental.pallas import tpu as pltpu
from jax.experimental.pallas import tpu_sc as plsc

B = 256
T = 50
H = 64
EMB = 32
K = 20
SIZE = 100000
CHUNK = 2048
NCHUNK = (SIZE + CHUNK - 1) // CHUNK
PAD_SIZE = NCHUNK * CHUNK
NEG = -3.0e38


def _gru_body(x_ref, wih0_ref, whh0_ref, wih1_ref, whh1_ref, b0_ref, b1_ref,
              out_ref):
    wih0 = wih0_ref[...]
    whh0 = whh0_ref[...]
    wih1 = wih1_ref[...]
    whh1 = whh1_ref[...]
    b_ih0 = b0_ref[0:1, :]
    b_hh0 = b0_ref[1:2, :]
    b_ih1 = b1_ref[0:1, :]
    b_hh1 = b1_ref[1:2, :]

    def gru_cell(gx, h, whh, b_hh):
        gh = jnp.dot(h, whh, preferred_element_type=jnp.float32) + b_hh
        def gates(side):
            o = 3 * H * side
            r = jax.nn.sigmoid(gx[:, o:o + H] + gh[:, o:o + H])
            z = jax.nn.sigmoid(gx[:, o + H:o + 2 * H] + gh[:, o + H:o + 2 * H])
            n = jnp.tanh(gx[:, o + 2 * H:o + 3 * H]
                         + r * gh[:, o + 2 * H:o + 3 * H])
            hs = h[:, H * side:H * (side + 1)]
            return (1.0 - z) * n + z * hs
        return jnp.concatenate([gates(0), gates(1)], axis=1)

    x = x_ref[...]
    iota_r = jax.lax.broadcasted_iota(jnp.int32, (2 * T, 2), 0)
    iota_c = jax.lax.broadcasted_iota(jnp.int32, (2 * T, 2), 1)

    def step(t, carry):
        h1, h2 = carry
        oh = (iota_r == t + T * iota_c).astype(jnp.float32)
        xt = jnp.dot(x, oh, preferred_element_type=jnp.float32)
        gx0 = jnp.dot(xt, wih0, preferred_element_type=jnp.float32) + b_ih0
        h1 = gru_cell(gx0, h1, whh0, b_hh0)
        gx1 = jnp.dot(h1, wih1, preferred_element_type=jnp.float32) + b_ih1
        h2 = gru_cell(gx1, h2, whh1, b_hh1)
        return (h1, h2)

    h0 = jnp.zeros((B, 2 * H), jnp.float32)
    _, h2 = lax.fori_loop(0, T, step, (h0, h0))
    out_ref[...] = h2


def _run_gru(x2, wih0, whh0, wih1, whh1, b0, b1):
    return pl.pallas_call(
        _gru_body,
        out_shape=jax.ShapeDtypeStruct((B, 2 * H), jnp.float32),
    )(x2, wih0, whh0, wih1, whh1, b0, b1)


def _topk_body(eblk_ref, off_ref, emb_ref, vals_ref, idxs_ref, s_ref, rv_ref,
               ri_ref):
    i = pl.program_id(0)

    @pl.when(i == 0)
    def _init():
        rv_ref[...] = jnp.full((B, 128), NEG, jnp.float32)
        ri_ref[...] = jnp.zeros((B, 128), jnp.int32)

    eb = eblk_ref[...]
    off = off_ref[...]
    e = jnp.zeros((B, EMB), jnp.float32)
    for kq in range(4):
        e = e + jnp.where(off == 32 * kq, eb[:, 32 * kq:32 * (kq + 1)], 0.0)
    emb_c = emb_ref[...]
    na = jnp.sum(e * e, axis=1, keepdims=True)
    nb = jnp.sum(emb_c * emb_c, axis=0, keepdims=True)
    s2 = jnp.dot(e, emb_c, preferred_element_type=jnp.float32)
    d2 = jnp.maximum(na + nb - 2.0 * s2, 0.0)
    col = jax.lax.broadcasted_iota(jnp.int32, (B, CHUNK), 1) + i * CHUNK
    s_ref[...] = jnp.where(col < SIZE, -d2, NEG)

    iota_c = jax.lax.broadcasted_iota(jnp.int32, (B, CHUNK), 1)
    iota_r = jax.lax.broadcasted_iota(jnp.int32, (B, 128), 1)
    big = jnp.int32(2 ** 30)
    K1 = K + 1

    @pl.when(i == 0)
    def _first_chunk():
        vals = []
        idxs = []
        for _ in range(K1):
            s = s_ref[...]
            m_s = jnp.max(s, axis=1, keepdims=True)
            c_s = jnp.min(jnp.where(s == m_s, iota_c, big), axis=1,
                          keepdims=True)
            s_ref[...] = jnp.where(iota_c == c_s, NEG, s)
            vals.append(m_s)
            idxs.append(c_s)
        pad_v = jnp.full((B, 128 - K1), NEG, jnp.float32)
        pad_i = jnp.zeros((B, 128 - K1), jnp.int32)
        rv_ref[...] = jnp.concatenate(vals + [pad_v], axis=1)
        ri_ref[...] = jnp.concatenate(idxs + [pad_i], axis=1)

    @pl.when(i > 0)
    def _merge_chunk():
        s0 = s_ref[...]
        theta0 = jnp.min(rv_ref[...][:, :K1], axis=1, keepdims=True)
        cnt = jnp.sum(jnp.where(s0 > theta0, 1, 0), axis=1, keepdims=True)
        trip = jnp.max(cnt)

        def insert(_, carry):
            s = s_ref[...]
            rv = rv_ref[...]
            ri = ri_ref[...]
            rv21 = rv[:, :K1]
            theta = jnp.min(rv21, axis=1, keepdims=True)
            c_min = jnp.min(jnp.where(rv21 == theta,
                                      iota_r[:, :K1], big),
                            axis=1, keepdims=True)
            m_s = jnp.max(s, axis=1, keepdims=True)
            take = m_s > theta
            c_s = jnp.min(jnp.where(s == m_s, iota_c, big), axis=1,
                          keepdims=True)
            hit = (iota_r == c_min) & take
            rv_ref[...] = jnp.where(hit, m_s, rv)
            ri_ref[...] = jnp.where(hit, c_s + i * CHUNK, ri)
            s_ref[...] = jnp.where((iota_c == c_s) & take, NEG, s)
            return carry

        lax.fori_loop(0, trip, insert, 0)

    @pl.when(i == NCHUNK - 1)
    def _emit():
        vals_ref[...] = rv_ref[...][:, :K + 1]
        idxs_ref[...] = ri_ref[...][:, :K + 1]


def _run_topk(e_blk, off4, emb_pad):
    return pl.pallas_call(
        _topk_body,
        grid=(NCHUNK,),
        in_specs=[
            pl.BlockSpec((B, 128), lambda i: (0, 0)),
            pl.BlockSpec((B, 1), lambda i: (0, 0)),
            pl.BlockSpec((EMB, CHUNK), lambda i: (0, i)),
        ],
        out_specs=[
            pl.BlockSpec((B, K + 1), lambda i: (0, 0)),
            pl.BlockSpec((B, K + 1), lambda i: (0, 0)),
        ],
        out_shape=[
            jax.ShapeDtypeStruct((B, K + 1), jnp.float32),
            jax.ShapeDtypeStruct((B, K + 1), jnp.int32),
        ],
        scratch_shapes=[
            pltpu.VMEM((B, CHUNK), jnp.float32),
            pltpu.VMEM((B, 128), jnp.float32),
            pltpu.VMEM((B, 128), jnp.int32),
        ],
    )(e_blk, off4, emb_pad)


_NW = 32
_BPW = B // _NW
NIDX = 24
_IPW = B * NIDX // _NW


def _sc_mesh():
    return plsc.VectorSubcoreMesh(core_axis_name="c", subcore_axis_name="s")


def _embsel_body(emb1, bidx1, emb2, bidx2, o1, o2, bidx_v, rows_v, sem):
    wid = lax.axis_index("s") * 2 + lax.axis_index("c")
    for emb, bidx, o in ((emb1, bidx1, o1), (emb2, bidx2, o2)):
        pltpu.sync_copy(bidx.at[pl.ds(wid * _BPW, _BPW)], bidx_v)
        pltpu.async_copy(emb.at[bidx_v], rows_v, sem).wait()
        pltpu.sync_copy(rows_v, o.at[pl.ds(wid * _BPW, _BPW)])


def _run_embsel(emb1_128, bidx1, emb2_128, bidx2):
    f = pl.kernel(
        _embsel_body, mesh=_sc_mesh(),
        out_type=[jax.ShapeDtypeStruct((B, 128), jnp.float32),
                  jax.ShapeDtypeStruct((B, 128), jnp.float32)],
        scratch_types=[pltpu.VMEM((_BPW,), jnp.int32),
                       pltpu.VMEM((_BPW, 128), jnp.float32),
                       pltpu.SemaphoreType.DMA],
    )
    return f(emb1_128, bidx1, emb2_128, bidx2)


def _ygather_body(y1, rows1, y2, rows2, o1, o2, ridx_v, rows_v, sem):
    wid = lax.axis_index("s") * 2 + lax.axis_index("c")
    base = wid * _IPW
    for y, rows, o in ((y1, rows1, o1), (y2, rows2, o2)):
        pltpu.sync_copy(rows.at[wid], ridx_v)
        for h in range(2):
            pltpu.async_copy(y.at[ridx_v.at[h]],
                             rows_v.at[pl.ds(96 * h, 96)], sem).wait()
        pltpu.sync_copy(rows_v, o.at[pl.ds(base, _IPW)])


def _run_ygather(y1_128, rows1, y2_128, rows2):
    f = pl.kernel(
        _ygather_body, mesh=_sc_mesh(),
        out_type=[jax.ShapeDtypeStruct((B * NIDX, 128), jnp.float32),
                  jax.ShapeDtypeStruct((B * NIDX, 128), jnp.float32)],
        scratch_types=[pltpu.VMEM((2, _IPW // 2), jnp.int32),
                       pltpu.VMEM((_IPW, 128), jnp.float32),
                       pltpu.SemaphoreType.DMA],
    )
    return f(y1_128, rows1, y2_128, rows2)


def _final_body(hh_ref, v1_ref, v2_ref, yb1_ref, ln1_ref, yb2_ref, ln2_ref,
                wmean_ref, wstd_ref, wout_ref, bout_ref, wmo_ref, wso_ref,
                bs_ref, y_ref, err1_ref, err2_ref, mo_ref):
    temp = hh_ref[...]
    mean_ts = jnp.dot(temp, wmean_ref[...],
                      preferred_element_type=jnp.float32) + bs_ref[0, 0]
    std_ts = jnp.dot(temp, wstd_ref[...],
                     preferred_element_type=jnp.float32) + bs_ref[0, 1]

    iota3 = jax.lax.broadcasted_iota(jnp.int32, (B, NIDX, 128), 2)
    iota21 = jax.lax.broadcasted_iota(jnp.int32, (B, K + 1), 1)
    big = jnp.int32(2 ** 30)

    def feats_one(v_ref, yb_ref, ln_ref):
        v = v_ref[...]
        d2 = -v
        w = jnp.exp(-jnp.where(d2 > 0, jnp.sqrt(jnp.where(d2 > 0, d2, 1.0)),
                               0.0))
        vmax = jnp.max(v, axis=1, keepdims=True)
        c_self = jnp.min(jnp.where(v == vmax, iota21, big), axis=1,
                         keepdims=True)
        m = jnp.where(iota21 == c_self, 0.0, 1.0)
        oh = (iota3 == ln_ref[...]).astype(jnp.float32)
        ys = jnp.sum(yb_ref[...] * oh, axis=2)
        sel = ys[:, :K + 1]
        wsel = w * m
        ws = jnp.sum(wsel, axis=1, keepdims=True)
        wm = jnp.sum(sel * wsel, axis=1, keepdims=True) / ws
        mu = jnp.sum(sel * m, axis=1, keepdims=True) / K
        st = jnp.sqrt(jnp.sum(m * (sel - mu) ** 2, axis=1, keepdims=True)
                      / (K - 1))
        return jnp.concatenate([wm, ws, st], axis=1)

    f1 = feats_one(v1_ref, yb1_ref, ln1_ref)
    f2 = feats_one(v2_ref, yb2_ref, ln2_ref)
    feats = jnp.concatenate([f1, f2, mean_ts, std_ts], axis=1)
    h = jnp.maximum(jnp.dot(feats, wout_ref[...],
                            preferred_element_type=jnp.float32)
                    + bout_ref[...], 0.0)
    mean_o = jnp.dot(h, wmo_ref[...],
                     preferred_element_type=jnp.float32) + bs_ref[0, 2]
    std_o = jnp.dot(h, wso_ref[...],
                    preferred_element_type=jnp.float32) + bs_ref[0, 3]

    yb = y_ref[...]
    e1 = ((yb - mean_ts) ** 2 / jnp.exp(std_ts)) + std_ts
    e2 = ((yb - mean_o) ** 2 / jnp.exp(std_o)) + std_o
    err1_ref[...] = jnp.sum(e1, axis=0, keepdims=True) / B
    err2_ref[...] = jnp.sum(e2, axis=0, keepdims=True) / B
    mo_ref[...] = mean_o


def _run_final(hh, v1, v2, yb1, ln1, yb2, ln2, wmean, wstd, wout, bout, wmo,
               wso, bs, y):
    return pl.pallas_call(
        _final_body,
        out_shape=[
            jax.ShapeDtypeStruct((1, 1), jnp.float32),
            jax.ShapeDtypeStruct((1, 1), jnp.float32),
            jax.ShapeDtypeStruct((B, 1), jnp.float32),
        ],
    )(hh, v1, v2, yb1, ln1, yb2, ln2, wmean, wstd, wout, bout, wmo,
      wso, bs, y)


def kernel(params, x_left, x_right, y, index1, index2, y1_context, y2_context):
    p = params
    x2 = jnp.concatenate([x_left, x_right], axis=1)

    def _bd(a, b):
        ka, kb = a.shape[0], b.shape[0]
        z1 = jnp.zeros((ka, 3 * H), jnp.float32)
        z2 = jnp.zeros((kb, 3 * H), jnp.float32)
        return jnp.concatenate([jnp.concatenate([a, z1], 1),
                                jnp.concatenate([z2, b], 1)], 0)
    wih0 = _bd(p['W_ih_left0'][:, 0][None, :], p['W_ih_right0'][:, 0][None, :])
    whh0 = _bd(p['W_hh_left0'].T, p['W_hh_right0'].T)
    wih1 = _bd(p['W_ih_left1'].T, p['W_ih_right1'].T)
    whh1 = _bd(p['W_hh_left1'].T, p['W_hh_right1'].T)
    b0 = jnp.stack([jnp.concatenate([p['b_ih_left0'], p['b_ih_right0']]),
                    jnp.concatenate([p['b_hh_left0'], p['b_hh_right0']])])
    b1 = jnp.stack([jnp.concatenate([p['b_ih_left1'], p['b_ih_right1']]),
                    jnp.concatenate([p['b_hh_left1'], p['b_hh_right1']])])
    hh = _run_gru(x2, wih0, whh0, wih1, whh1, b0, b1)

    idx1i = index1.astype(jnp.int32)
    idx2i = index2.astype(jnp.int32)
    eb1, eb2 = _run_embsel(p['emb1'].reshape(-1, 128), idx1i // 4,
                           p['emb2'].reshape(-1, 128), idx2i // 4)
    off1 = ((idx1i % 4) * EMB)[:, None]
    off2 = ((idx2i % 4) * EMB)[:, None]

    emb1_pad = jnp.pad(p['emb1'].T, ((0, 0), (0, PAD_SIZE - SIZE)))
    emb2_pad = jnp.pad(p['emb2'].T, ((0, 0), (0, PAD_SIZE - SIZE)))
    v1, i1 = _run_topk(eb1, off1, emb1_pad)
    v2, i2 = _run_topk(eb2, off2, emb2_pad)

    bb = jnp.arange(B, dtype=jnp.int32)[:, None] * SIZE
    def _yidx(ii):
        fi = jnp.pad(bb + ii, ((0, 0), (0, NIDX - (K + 1))))
        rows = (fi // 128).reshape(_NW, 2, _IPW // 2)
        lanes = (fi % 128).reshape(B, NIDX, 1)
        return rows, lanes
    rows1, lanes1 = _yidx(i1)
    rows2, lanes2 = _yidx(i2)
    yb1, yb2 = _run_ygather(
        y1_context.reshape(B * SIZE // 128, 128), rows1,
        y2_context.reshape(B * SIZE // 128, 128), rows2)
    yb1 = yb1.reshape(B, NIDX, 128)
    yb2 = yb2.reshape(B, NIDX, 128)

    bs = jnp.stack([p['b_mean'][0], p['b_std'][0],
                    p['b_mo'][0], p['b_so'][0]])[None, :]
    err1, err2, mean_o = _run_final(
        hh, v1, v2, yb1, lanes1, yb2, lanes2,
        p['W_mean'].T, p['W_std'].T, p['W_out1'].T, p['b_out1'][None, :],
        p['W_mo'].T, p['W_so'].T, bs, y[:, None])
    return (err1.reshape(()), err2.reshape(()), mean_o)
